# Initial kernel scaffold; baseline (speedup 1.0000x reference)
import functools

import jax
import jax.numpy as jnp
from jax import lax
from jax.experimental import pallas as pl
from jax.experimental.pallas import tpu as pltpu
from jax.experimental.pallas import tpu_sc as plsc

VOCAB = 1000000
EMBED = 64
BATCH = 16384
SEQ = 50

NC = 2
NS = 16
NW = NC * NS

TOTAL = BATCH * SEQ
PER_W = TOTAL // NW
CHUNK = 128
NCHUNK = PER_W // CHUNK


@functools.partial(
    pl.kernel,
    out_type=(
        jax.ShapeDtypeStruct((TOTAL, EMBED), jnp.float32),
        jax.ShapeDtypeStruct((TOTAL, EMBED), jnp.float32),
    ),
    mesh=plsc.VectorSubcoreMesh(core_axis_name="c", subcore_axis_name="s"),
    scratch_types=[
        pltpu.VMEM((NCHUNK, CHUNK), jnp.int32),
        pltpu.VMEM((CHUNK, EMBED), jnp.float32),
        pltpu.SemaphoreType.DMA,
    ],
)
def _embed_lookup(src_table, src_idx, tgt_table, tgt_idx,
                  src_out, tgt_out, idx_v, rows_v, sem):
    wid = lax.axis_index("s") * NC + lax.axis_index("c")
    row_base = wid * PER_W
    chunk_base = wid * NCHUNK

    def run_table(table, idx_hbm, out):
        pltpu.sync_copy(idx_hbm.at[pl.ds(chunk_base, NCHUNK)], idx_v)

        def body(j, carry):
            pltpu.async_copy(table.at[idx_v.at[j]], rows_v, sem).wait()
            pltpu.sync_copy(rows_v, out.at[pl.ds(row_base + j * CHUNK, CHUNK)])
            return carry

        lax.fori_loop(0, NCHUNK, body, 0)

    run_table(src_table, src_idx, src_out)
    run_table(tgt_table, tgt_idx, tgt_out)


def kernel(src_indices, tgt_indices, src_table, tgt_table):
    src_idx = src_indices.reshape(-1, CHUNK).astype(jnp.int32)
    tgt_idx = tgt_indices.reshape(-1, CHUNK).astype(jnp.int32)
    src_out, tgt_out = _embed_lookup(src_table, src_idx, tgt_table, tgt_idx)
    return (
        src_out.reshape(BATCH, SEQ, EMBED),
        tgt_out.reshape(BATCH, SEQ, EMBED),
    )


# speedup vs baseline: 1.7261x; 1.7261x over previous
import functools

import jax
import jax.numpy as jnp
from jax import lax
from jax.experimental import pallas as pl
from jax.experimental.pallas import tpu as pltpu
from jax.experimental.pallas import tpu_sc as plsc

VOCAB = 1000000
EMBED = 64
BATCH = 16384
SEQ = 50

NC = 2
NS = 16
NW = NC * NS

TOTAL = BATCH * SEQ
PER_W = TOTAL // NW
CHUNK = 128
NCHUNK = PER_W // CHUNK


@functools.partial(
    pl.kernel,
    out_type=(
        jax.ShapeDtypeStruct((TOTAL, EMBED), jnp.float32),
        jax.ShapeDtypeStruct((TOTAL, EMBED), jnp.float32),
    ),
    mesh=plsc.VectorSubcoreMesh(core_axis_name="c", subcore_axis_name="s"),
    scratch_types=[
        pltpu.VMEM((NCHUNK, CHUNK), jnp.int32),
        pltpu.VMEM((CHUNK, EMBED), jnp.float32),
        pltpu.SemaphoreType.DMA,
    ],
    compiler_params=pltpu.CompilerParams(use_tc_tiling_on_sc=False),
)
def _embed_lookup(src_table, src_idx, tgt_table, tgt_idx,
                  src_out, tgt_out, idx_v, rows_v, sem):
    wid = lax.axis_index("s") * NC + lax.axis_index("c")
    row_base = wid * PER_W
    chunk_base = wid * NCHUNK

    def run_table(table, idx_hbm, out):
        pltpu.sync_copy(idx_hbm.at[pl.ds(chunk_base, NCHUNK)], idx_v)

        def body(j, carry):
            pltpu.async_copy(table.at[idx_v.at[j]], rows_v, sem).wait()
            pltpu.sync_copy(rows_v, out.at[pl.ds(row_base + j * CHUNK, CHUNK)])
            return carry

        lax.fori_loop(0, NCHUNK, body, 0)

    run_table(src_table, src_idx, src_out)
    run_table(tgt_table, tgt_idx, tgt_out)


def kernel(src_indices, tgt_indices, src_table, tgt_table):
    src_idx = src_indices.reshape(-1, CHUNK).astype(jnp.int32)
    tgt_idx = tgt_indices.reshape(-1, CHUNK).astype(jnp.int32)
    src_out, tgt_out = _embed_lookup(src_table, src_idx, tgt_table, tgt_idx)
    return (
        src_out.reshape(BATCH, SEQ, EMBED),
        tgt_out.reshape(BATCH, SEQ, EMBED),
    )


# speedup vs baseline: 1.9414x; 1.1247x over previous
import functools

import jax
import jax.numpy as jnp
from jax import lax
from jax.experimental import pallas as pl
from jax.experimental.pallas import tpu as pltpu
from jax.experimental.pallas import tpu_sc as plsc

VOCAB = 1000000
EMBED = 64
BATCH = 16384
SEQ = 50

NC = 2
NS = 16
NW = NC * NS

TOTAL = BATCH * SEQ
PER_W = TOTAL // NW
CHUNK = 128
NCHUNK = PER_W // CHUNK

NB = 8
DEPTH = 4
GROUPS = (NCHUNK - 2 * DEPTH) // NB


@functools.partial(
    pl.kernel,
    out_type=(
        jax.ShapeDtypeStruct((TOTAL, EMBED), jnp.float32),
        jax.ShapeDtypeStruct((TOTAL, EMBED), jnp.float32),
    ),
    mesh=plsc.VectorSubcoreMesh(core_axis_name="c", subcore_axis_name="s"),
    scratch_types=[
        pltpu.VMEM((NCHUNK, CHUNK), jnp.int32),
        pltpu.VMEM((NB, CHUNK, EMBED), jnp.float32),
        pltpu.SemaphoreType.DMA((NB,)),
        pltpu.SemaphoreType.DMA((NB,)),
    ],
    compiler_params=pltpu.CompilerParams(use_tc_tiling_on_sc=False),
)
def _embed_lookup(src_table, src_idx, tgt_table, tgt_idx,
                  src_out, tgt_out, idx_v, rows, gsem, ssem):
    wid = lax.axis_index("s") * NC + lax.axis_index("c")
    row_base = wid * PER_W
    chunk_base = wid * NCHUNK

    def run_table(table, idx_hbm, out):
        pltpu.sync_copy(idx_hbm.at[pl.ds(chunk_base, NCHUNK)], idx_v)

        def fire_gather(b, j):
            pltpu.async_copy(table.at[idx_v.at[j]], rows.at[b], gsem.at[b])

        def wait_gather(b):
            pltpu.make_async_copy(
                table.at[idx_v.at[0]], rows.at[b], gsem.at[b]).wait()

        def fire_store(b, j):
            pltpu.async_copy(
                rows.at[b], out.at[pl.ds(row_base + j * CHUNK, CHUNK)],
                ssem.at[b])

        def wait_store(b):
            pltpu.make_async_copy(
                rows.at[b], out.at[pl.ds(row_base, CHUNK)], ssem.at[b]).wait()

        for b in range(DEPTH):
            fire_gather(b, b)
        for t in range(DEPTH):
            wait_gather(t)
            fire_store(t, t)
            fire_gather(t + DEPTH, t + DEPTH)

        def body(g, carry):
            j0 = DEPTH + g * NB
            for t in range(NB):
                j = j0 + t
                b_new = t
                b_cur = (t + DEPTH) % NB
                wait_store(b_new)
                fire_gather(b_new, j + DEPTH)
                wait_gather(b_cur)
                fire_store(b_cur, j)
            return carry

        lax.fori_loop(0, GROUPS, body, 0)

        for t in range(DEPTH):
            j = NCHUNK - DEPTH + t
            b = j % NB
            wait_gather(b)
            fire_store(b, j)
        for b in range(NB):
            wait_store(b)

    run_table(src_table, src_idx, src_out)
    run_table(tgt_table, tgt_idx, tgt_out)


def kernel(src_indices, tgt_indices, src_table, tgt_table):
    src_idx = src_indices.reshape(-1, CHUNK).astype(jnp.int32)
    tgt_idx = tgt_indices.reshape(-1, CHUNK).astype(jnp.int32)
    src_out, tgt_out = _embed_lookup(src_table, src_idx, tgt_table, tgt_idx)
    return (
        src_out.reshape(BATCH, SEQ, EMBED),
        tgt_out.reshape(BATCH, SEQ, EMBED),
    )


# speedup vs baseline: 2.0130x; 1.0369x over previous
import functools

import jax
import jax.numpy as jnp
from jax import lax
from jax.experimental import pallas as pl
from jax.experimental.pallas import tpu as pltpu
from jax.experimental.pallas import tpu_sc as plsc

VOCAB = 1000000
EMBED = 64
BATCH = 16384
SEQ = 50

NC = 2
NS = 16
NW = NC * NS

TOTAL = BATCH * SEQ
PER_W = TOTAL // NW
CHUNK = 128
NCHUNK = PER_W // CHUNK

NB = 8
DEPTH = 4
GROUPS = (NCHUNK - 2 * DEPTH) // NB


@functools.partial(
    pl.kernel,
    out_type=jax.ShapeDtypeStruct((TOTAL, EMBED), jnp.float32),
    mesh=plsc.VectorSubcoreMesh(core_axis_name="c", subcore_axis_name="s"),
    scratch_types=[
        pltpu.VMEM((NCHUNK, CHUNK), jnp.int32),
        pltpu.VMEM((NB, CHUNK, EMBED), jnp.float32),
        pltpu.SemaphoreType.DMA((NB,)),
        pltpu.SemaphoreType.DMA((NB,)),
    ],
    compiler_params=pltpu.CompilerParams(use_tc_tiling_on_sc=False),
)
def _embed_lookup(table, idx_hbm, out, idx_v, rows, gsem, ssem):
    wid = lax.axis_index("s") * NC + lax.axis_index("c")
    row_base = wid * PER_W
    chunk_base = wid * NCHUNK

    pltpu.sync_copy(idx_hbm.at[pl.ds(chunk_base, NCHUNK)], idx_v)

    def fire_gather(b, j):
        pltpu.async_copy(table.at[idx_v.at[j]], rows.at[b], gsem.at[b])

    def wait_gather(b):
        pltpu.make_async_copy(
            table.at[idx_v.at[0]], rows.at[b], gsem.at[b]).wait()

    def fire_store(b, j):
        pltpu.async_copy(
            rows.at[b], out.at[pl.ds(row_base + j * CHUNK, CHUNK)],
            ssem.at[b])

    def wait_store(b):
        pltpu.make_async_copy(
            rows.at[b], out.at[pl.ds(row_base, CHUNK)], ssem.at[b]).wait()

    for b in range(DEPTH):
        fire_gather(b, b)
    for t in range(DEPTH):
        wait_gather(t)
        fire_store(t, t)
        fire_gather(t + DEPTH, t + DEPTH)

    def body(g, carry):
        j0 = DEPTH + g * NB
        for t in range(NB):
            j = j0 + t
            b_new = t
            b_cur = (t + DEPTH) % NB
            wait_store(b_new)
            fire_gather(b_new, j + DEPTH)
            wait_gather(b_cur)
            fire_store(b_cur, j)
        return carry

    lax.fori_loop(0, GROUPS, body, 0)

    for t in range(DEPTH):
        j = NCHUNK - DEPTH + t
        b = j % NB
        wait_gather(b)
        fire_store(b, j)
    for b in range(NB):
        wait_store(b)


def kernel(src_indices, tgt_indices, src_table, tgt_table):
    src_idx = src_indices.reshape(-1, CHUNK).astype(jnp.int32)
    tgt_idx = tgt_indices.reshape(-1, CHUNK).astype(jnp.int32)
    src_out = _embed_lookup(src_table, src_idx)
    tgt_out = _embed_lookup(tgt_table, tgt_idx)
    return (
        src_out.reshape(BATCH, SEQ, EMBED),
        tgt_out.reshape(BATCH, SEQ, EMBED),
    )
